# baseline (speedup 1.0000x reference)
import functools

import jax
import jax.numpy as jnp
from jax import lax
from jax.experimental import pallas as pl
from jax.experimental.pallas import tpu as pltpu
from jax.experimental.pallas import tpu_sc as plsc

_NUM_WORKERS = 32
_CHUNK = 128


def _sc_gather_concat(fans2, avg2, idx_all, b, d):
    f_out = 2 * d
    bpw = b // _NUM_WORKERS
    nch = bpw // _CHUNK
    seg = nch * 8 * _CHUNK
    mesh = plsc.VectorSubcoreMesh(core_axis_name="c", subcore_axis_name="s")

    @functools.partial(
        pl.kernel,
        mesh=mesh,
        compiler_params=pltpu.CompilerParams(needs_layout_passes=False),
        out_type=jax.ShapeDtypeStruct((b * f_out,), jnp.float32),
        scratch_types=[
            pltpu.VMEM((2 * nch, _CHUNK), jnp.int32),
            pltpu.VMEM((2 * nch, _CHUNK), jnp.int32),
            pltpu.VMEM((_CHUNK, 128), jnp.float32),
            pltpu.VMEM((_CHUNK, 128), jnp.float32),
            pltpu.VMEM((f_out // 8 * seg,), jnp.float32),
            pltpu.SemaphoreType.DMA,
            pltpu.SemaphoreType.DMA,
        ],
    )
    def k(fans_hbm, avg_hbm, idx_hbm, out_hbm,
          idx_v, ridx, raw_a, raw_b, tp, sem_a, sem_b):
        wid = lax.axis_index("s") * 2 + lax.axis_index("c")
        pltpu.sync_copy(idx_hbm.at[wid], idx_v)
        for c in range(2 * nch):
            for g in range(8):
                sl = pl.ds(g * 16, 16)
                ridx[c, sl] = idx_v[c, sl] >> 2

        bufs = (raw_a, raw_b)
        sems = (sem_a, sem_b)
        tables = [fans_hbm] * nch + [avg_hbm] * nch
        iota = lax.iota(jnp.int32, 16)

        def fire(c):
            return pltpu.async_copy(
                tables[c].at[ridx.at[c]], bufs[c % 2], sems[c % 2])

        inflight = fire(0)
        for c in range(2 * nch):
            nxt = fire(c + 1) if c + 1 < 2 * nch else None
            inflight.wait()
            inflight = nxt

            raw = bufs[c % 2]
            kk = c % nch
            foff = (c // nch) * d
            rows = []
            lanes = []
            for g in range(8):
                iv = idx_v[c, pl.ds(g * 16, 16)]
                rows.append(iota + g * 16)
                lanes.append((iv & 3) * 32)

            @pl.loop(0, d)
            def _f(f, kk=kk, foff=foff, rows=rows, lanes=lanes, raw=raw):
                fo = f + foff
                toff = (fo >> 3) * seg + kk * (8 * _CHUNK) + (fo & 7) * _CHUNK
                fb = jnp.full((16,), f, jnp.int32)
                for g in range(8):
                    vals = plsc.load_gather(raw, [rows[g], lanes[g] + fb])
                    tp[pl.ds(toff + g * 16, 16)] = vals

        out_copies = []
        for tr in range(f_out // 8):
            out_copies.append(pltpu.async_copy(
                tp.at[pl.ds(tr * seg, seg)],
                out_hbm.at[pl.ds(tr * (8 * b) + wid * seg, seg)],
                sem_a))
        for cp in out_copies:
            cp.wait()

    return k(fans2, avg2, idx_all)


def kernel(user_fea, fans_table, avgrating_table):
    b = user_fea.shape[0]
    d = fans_table.shape[1]
    bpw = b // _NUM_WORKERS
    nch = bpw // _CHUNK
    fans2 = fans_table.reshape(fans_table.shape[0] * d // 128, 128)
    avg2 = avgrating_table.reshape(avgrating_table.shape[0] * d // 128, 128)
    fidx = user_fea[:, 1].astype(jnp.int32).reshape(_NUM_WORKERS, nch, _CHUNK)
    aidx = user_fea[:, 2].astype(jnp.int32).reshape(_NUM_WORKERS, nch, _CHUNK)
    idx_all = jnp.concatenate((fidx, aidx), axis=1)
    y = _sc_gather_concat(fans2, avg2, idx_all, b, d)
    y = y.reshape(2 * d // 8, b // _CHUNK, 8, _CHUNK)
    return y.transpose(1, 3, 0, 2).reshape(b, 2 * d)


# speedup vs baseline: 1.0115x; 1.0115x over previous
import functools

import jax
import jax.numpy as jnp
from jax import lax
from jax.experimental import pallas as pl
from jax.experimental.pallas import tpu as pltpu
from jax.experimental.pallas import tpu_sc as plsc

_NUM_WORKERS = 32
_CHUNK = 128


def _sc_gather_concat(fans_pad, avg_pad, idx_all, b, d):
    f_out = 2 * d
    bpw = b // _NUM_WORKERS
    nch = bpw // _CHUNK
    seg = nch * 8 * _CHUNK
    mesh = plsc.VectorSubcoreMesh(core_axis_name="c", subcore_axis_name="s")

    @functools.partial(
        pl.kernel,
        mesh=mesh,
        compiler_params=pltpu.CompilerParams(needs_layout_passes=False),
        out_type=jax.ShapeDtypeStruct((b * f_out,), jnp.float32),
        scratch_types=[
            pltpu.VMEM((2 * nch, _CHUNK), jnp.int32),
            pltpu.VMEM((_CHUNK, 128), jnp.float32),
            pltpu.VMEM((_CHUNK, 128), jnp.float32),
            pltpu.VMEM((f_out // 8 * seg,), jnp.float32),
            pltpu.SemaphoreType.DMA,
            pltpu.SemaphoreType.DMA,
        ],
    )
    def k(fans_hbm, avg_hbm, idx_hbm, out_hbm,
          idx_v, raw_a, raw_b, tp, sem_a, sem_b):
        wid = lax.axis_index("s") * 2 + lax.axis_index("c")
        pltpu.sync_copy(idx_hbm.at[wid], idx_v)

        bufs = (raw_a, raw_b)
        sems = (sem_a, sem_b)
        tables = [fans_hbm] * nch + [avg_hbm] * nch
        iota = lax.iota(jnp.int32, 16)
        rows = [iota + g * 16 for g in range(8)]

        def fire(c):
            return pltpu.async_copy(
                tables[c].at[idx_v.at[c]], bufs[c % 2], sems[c % 2])

        inflight = fire(0)
        for c in range(2 * nch):
            nxt = fire(c + 1) if c + 1 < 2 * nch else None
            inflight.wait()
            inflight = nxt

            raw = bufs[c % 2]
            kk = c % nch
            foff = (c // nch) * d

            @pl.loop(0, d)
            def _f(f, kk=kk, foff=foff, raw=raw):
                fo = f + foff
                toff = (fo >> 3) * seg + kk * (8 * _CHUNK) + (fo & 7) * _CHUNK
                fb = jnp.full((16,), f, jnp.int32)
                for g in range(8):
                    vals = plsc.load_gather(raw, [rows[g], fb])
                    tp[pl.ds(toff + g * 16, 16)] = vals

        out_copies = []
        for tr in range(f_out // 8):
            out_copies.append(pltpu.async_copy(
                tp.at[pl.ds(tr * seg, seg)],
                out_hbm.at[pl.ds(tr * (8 * b) + wid * seg, seg)],
                sem_a))
        for cp in out_copies:
            cp.wait()

    return k(fans_pad, avg_pad, idx_all)


def kernel(user_fea, fans_table, avgrating_table):
    b = user_fea.shape[0]
    d = fans_table.shape[1]
    bpw = b // _NUM_WORKERS
    nch = bpw // _CHUNK
    fans_pad = jnp.pad(fans_table, ((0, 0), (0, 128 - d)))
    avg_pad = jnp.pad(avgrating_table, ((0, 0), (0, 128 - d)))
    fidx = user_fea[:, 1].astype(jnp.int32).reshape(_NUM_WORKERS, nch, _CHUNK)
    aidx = user_fea[:, 2].astype(jnp.int32).reshape(_NUM_WORKERS, nch, _CHUNK)
    idx_all = jnp.concatenate((fidx, aidx), axis=1)
    y = _sc_gather_concat(fans_pad, avg_pad, idx_all, b, d)
    y = y.reshape(2 * d // 8, b // _CHUNK, 8, _CHUNK)
    return y.transpose(1, 3, 0, 2).reshape(b, 2 * d)


# speedup vs baseline: 1.0843x; 1.0719x over previous
import functools

import jax
import jax.numpy as jnp
from jax import lax
from jax.experimental import pallas as pl
from jax.experimental.pallas import tpu as pltpu
from jax.experimental.pallas import tpu_sc as plsc

_NUM_WORKERS = 32
_CHUNK = 128


def _sc_gather_concat(fans_table, avgrating_table, fidx, aidx):
    b = fidx.shape[0] * fidx.shape[1] * fidx.shape[2]
    d = fans_table.shape[1]
    bpw = b // _NUM_WORKERS
    nch = bpw // _CHUNK
    mesh = plsc.VectorSubcoreMesh(core_axis_name="c", subcore_axis_name="s")

    @functools.partial(
        pl.kernel,
        mesh=mesh,
        compiler_params=pltpu.CompilerParams(use_tc_tiling_on_sc=False),
        out_type=jax.ShapeDtypeStruct((b, 2 * d), jnp.float32),
        scratch_types=[
            pltpu.VMEM((nch, _CHUNK), jnp.int32),
            pltpu.VMEM((nch, _CHUNK), jnp.int32),
            pltpu.VMEM((bpw, d), jnp.float32),
            pltpu.VMEM((bpw, d), jnp.float32),
            pltpu.SemaphoreType.DMA,
        ],
    )
    def k(fans_hbm, avg_hbm, fidx_hbm, aidx_hbm, out_hbm,
          fidx_v, aidx_v, frows, arows, sem):
        wid = lax.axis_index("s") * 2 + lax.axis_index("c")
        base = wid * bpw
        pltpu.sync_copy(fidx_hbm.at[wid], fidx_v)
        pltpu.sync_copy(aidx_hbm.at[wid], aidx_v)
        copies = []
        for t in range(nch):
            sl = pl.ds(t * _CHUNK, _CHUNK)
            copies.append(
                pltpu.async_copy(fans_hbm.at[fidx_v.at[t]], frows.at[sl], sem))
            copies.append(
                pltpu.async_copy(avg_hbm.at[aidx_v.at[t]], arows.at[sl], sem))
        for c in copies:
            c.wait()
        pltpu.sync_copy(frows, out_hbm.at[pl.ds(base, bpw), pl.ds(0, d)])
        pltpu.sync_copy(arows, out_hbm.at[pl.ds(base, bpw), pl.ds(d, d)])

    return k(fans_table, avgrating_table, fidx, aidx)


def kernel(user_fea, fans_table, avgrating_table):
    b = user_fea.shape[0]
    d = fans_table.shape[1]
    bpw = b // _NUM_WORKERS
    nch = bpw // _CHUNK
    fidx = user_fea[:, 1].astype(jnp.int32).reshape(_NUM_WORKERS, nch, _CHUNK)
    aidx = user_fea[:, 2].astype(jnp.int32).reshape(_NUM_WORKERS, nch, _CHUNK)
    return _sc_gather_concat(fans_table, avgrating_table, fidx, aidx)
